# Initial kernel scaffold; baseline (speedup 1.0000x reference)
import jax
import jax.numpy as jnp
from jax.experimental import pallas as pl


def kernel(x, edge_index, W1, b1, W2, b2):
    raise NotImplementedError("write your pallas kernel here")


# speedup vs baseline: 28.6269x; 28.6269x over previous
import functools

import jax
import jax.numpy as jnp
from jax import lax
from jax.experimental import pallas as pl
from jax.experimental.pallas import tpu as pltpu
from jax.experimental.pallas import tpu_sc as plsc

N_NODES = 10000
N_EDGES = 320000
D_IN = 128
D_HID = 64
N_CLS = 16

NC = 2
NS = 16
NW = NC * NS
CHUNK = 128
CHUNKS_PER_TILE = 80
E_PAD = NW * CHUNKS_PER_TILE * CHUNK
N_PAD = 10240
ROWS_PER_TILE = N_PAD // NS
DEGW = 16


def _sc_mesh():
    return plsc.VectorSubcoreMesh(
        core_axis_name="c", subcore_axis_name="s", num_cores=NC, num_subcores=NS
    )


def _degree_sc(dst_all):
    @functools.partial(
        pl.kernel,
        mesh=_sc_mesh(),
        out_type=jax.ShapeDtypeStruct((NC, N_PAD, DEGW), jnp.float32),
        scratch_types=[
            pltpu.VMEM((CHUNKS_PER_TILE, CHUNK), jnp.int32),
            pltpu.VMEM((CHUNK, DEGW), jnp.float32),
            pltpu.VMEM((ROWS_PER_TILE, DEGW), jnp.float32),
            pltpu.VMEM_SHARED((N_PAD, DEGW), jnp.float32),
        ],
        compiler_params=pltpu.CompilerParams(use_tc_tiling_on_sc=False),
    )
    def deg_kernel(dst_hbm, out_hbm, idxv, onesv, slicev, acc_sh):
        c = lax.axis_index("c")
        s = lax.axis_index("s")
        wid = s * NC + c

        def fill_ones(j, carry):
            onesv[j, :] = jnp.ones((DEGW,), jnp.float32)
            return carry

        lax.fori_loop(0, CHUNK, fill_ones, 0)

        def fill_zero(i, carry):
            slicev[i, :] = jnp.zeros((DEGW,), jnp.float32)
            return carry

        lax.fori_loop(0, ROWS_PER_TILE, fill_zero, 0)
        pltpu.sync_copy(slicev, acc_sh.at[pl.ds(s * ROWS_PER_TILE, ROWS_PER_TILE)])
        plsc.subcore_barrier()

        pltpu.sync_copy(dst_hbm.at[wid], idxv)

        def body(j, carry):
            pltpu.sync_copy(onesv, acc_sh.at[idxv.at[j]], add=True)
            return carry

        lax.fori_loop(0, CHUNKS_PER_TILE, body, 0)
        plsc.subcore_barrier()

        pltpu.sync_copy(acc_sh.at[pl.ds(s * ROWS_PER_TILE, ROWS_PER_TILE)], slicev)
        pltpu.sync_copy(
            slicev, out_hbm.at[c, pl.ds(s * ROWS_PER_TILE, ROWS_PER_TILE)]
        )

    return deg_kernel(dst_all)


def _aggregate_sc(y, src_all, dst_all, d):
    @functools.partial(
        pl.kernel,
        mesh=_sc_mesh(),
        out_type=jax.ShapeDtypeStruct((NC, N_PAD, d), jnp.float32),
        scratch_types=[
            pltpu.VMEM((CHUNKS_PER_TILE, CHUNK), jnp.int32),
            pltpu.VMEM((CHUNKS_PER_TILE, CHUNK), jnp.int32),
            pltpu.VMEM((CHUNK, d), jnp.float32),
            pltpu.VMEM((ROWS_PER_TILE, d), jnp.float32),
            pltpu.VMEM_SHARED((N_PAD, d), jnp.float32),
            pltpu.SemaphoreType.DMA,
        ],
        compiler_params=pltpu.CompilerParams(use_tc_tiling_on_sc=False),
    )
    def agg_kernel(y_hbm, src_hbm, dst_hbm, out_hbm, srcv, dstv, rows, slicev, acc_sh, sem):
        c = lax.axis_index("c")
        s = lax.axis_index("s")
        wid = s * NC + c

        def fill_zero(i, carry):
            for k in range(d // 16):
                slicev[i, pl.ds(k * 16, 16)] = jnp.zeros((16,), jnp.float32)
            return carry

        lax.fori_loop(0, ROWS_PER_TILE, fill_zero, 0)
        pltpu.sync_copy(slicev, acc_sh.at[pl.ds(s * ROWS_PER_TILE, ROWS_PER_TILE)])
        plsc.subcore_barrier()

        pltpu.sync_copy(src_hbm.at[wid], srcv)
        pltpu.sync_copy(dst_hbm.at[wid], dstv)

        def body(j, carry):
            pltpu.async_copy(y_hbm.at[srcv.at[j]], rows, sem).wait()
            pltpu.sync_copy(rows, acc_sh.at[dstv.at[j]], add=True)
            return carry

        lax.fori_loop(0, CHUNKS_PER_TILE, body, 0)
        plsc.subcore_barrier()

        pltpu.sync_copy(acc_sh.at[pl.ds(s * ROWS_PER_TILE, ROWS_PER_TILE)], slicev)
        pltpu.sync_copy(
            slicev, out_hbm.at[c, pl.ds(s * ROWS_PER_TILE, ROWS_PER_TILE)]
        )

    return agg_kernel(y, src_all, dst_all)


_R = 512


def _dinv_block(degp):
    deg = degp[0, :, 0:1] + degp[1, :, 0:1] + 1.0
    return lax.rsqrt(deg)


def _mm1_body(x_ref, w_ref, degp_ref, y_ref):
    dinv = _dinv_block(degp_ref[...])
    xw = jnp.dot(x_ref[...], w_ref[...], preferred_element_type=jnp.float32)
    y_ref[...] = xw * dinv


def _mid_body(p_ref, y1_ref, b1_ref, w2_ref, degp_ref, y2_ref):
    dinv = _dinv_block(degp_ref[...])
    p = p_ref[...]
    h = jnp.maximum((p[0] + p[1] + y1_ref[...]) * dinv + b1_ref[...], 0.0)
    y2_ref[...] = (
        jnp.dot(h, w2_ref[...], preferred_element_type=jnp.float32) * dinv
    )


def _out_body(p_ref, y2_ref, b2_ref, degp_ref, o_ref):
    dinv = _dinv_block(degp_ref[...])
    p = p_ref[...]
    z = (p[0] + p[1] + y2_ref[...]) * dinv + b2_ref[...]
    m = jnp.max(z, axis=1, keepdims=True)
    lse = jnp.log(jnp.sum(jnp.exp(z - m), axis=1, keepdims=True)) + m
    o_ref[...] = z - lse


def _tc_mm1(x_pad, W1, degp):
    return pl.pallas_call(
        _mm1_body,
        grid=(N_PAD // _R,),
        in_specs=[
            pl.BlockSpec((_R, D_IN), lambda i: (i, 0)),
            pl.BlockSpec((D_IN, D_HID), lambda i: (0, 0)),
            pl.BlockSpec((NC, _R, DEGW), lambda i: (0, i, 0)),
        ],
        out_specs=pl.BlockSpec((_R, D_HID), lambda i: (i, 0)),
        out_shape=jax.ShapeDtypeStruct((N_PAD, D_HID), jnp.float32),
    )(x_pad, W1, degp)


def _tc_mid(p1, y1, b1, W2, degp):
    return pl.pallas_call(
        _mid_body,
        grid=(N_PAD // _R,),
        in_specs=[
            pl.BlockSpec((NC, _R, D_HID), lambda i: (0, i, 0)),
            pl.BlockSpec((_R, D_HID), lambda i: (i, 0)),
            pl.BlockSpec((1, D_HID), lambda i: (0, 0)),
            pl.BlockSpec((D_HID, N_CLS), lambda i: (0, 0)),
            pl.BlockSpec((NC, _R, DEGW), lambda i: (0, i, 0)),
        ],
        out_specs=pl.BlockSpec((_R, N_CLS), lambda i: (i, 0)),
        out_shape=jax.ShapeDtypeStruct((N_PAD, N_CLS), jnp.float32),
    )(p1, y1, b1.reshape(1, D_HID), W2, degp)


def _tc_out(p2, y2, b2, degp):
    return pl.pallas_call(
        _out_body,
        grid=(N_PAD // _R,),
        in_specs=[
            pl.BlockSpec((NC, _R, N_CLS), lambda i: (0, i, 0)),
            pl.BlockSpec((_R, N_CLS), lambda i: (i, 0)),
            pl.BlockSpec((1, N_CLS), lambda i: (0, 0)),
            pl.BlockSpec((NC, _R, DEGW), lambda i: (0, i, 0)),
        ],
        out_specs=pl.BlockSpec((_R, N_CLS), lambda i: (i, 0)),
        out_shape=jax.ShapeDtypeStruct((N_PAD, N_CLS), jnp.float32),
    )(p2, y2, b2.reshape(1, N_CLS), degp)


def kernel(x, edge_index, W1, b1, W2, b2):
    ei = edge_index.astype(jnp.int32)
    src, dst = ei[0], ei[1]
    n_extra = E_PAD - N_EDGES
    pad_ar = jnp.arange(n_extra, dtype=jnp.int32)
    src_all = jnp.concatenate([src, pad_ar % N_NODES]).reshape(
        NW, CHUNKS_PER_TILE, CHUNK
    )
    dst_all = jnp.concatenate(
        [dst, N_NODES + pad_ar % (N_PAD - N_NODES)]
    ).reshape(NW, CHUNKS_PER_TILE, CHUNK)
    x_pad = jnp.pad(x, ((0, N_PAD - N_NODES), (0, 0)))

    degp = _degree_sc(dst_all)
    y1 = _tc_mm1(x_pad, W1, degp)
    p1 = _aggregate_sc(y1, src_all, dst_all, D_HID)
    y2 = _tc_mid(p1, y1, b1, W2, degp)
    p2 = _aggregate_sc(y2, src_all, dst_all, N_CLS)
    out = _tc_out(p2, y2, b2, degp)
    return out[:N_NODES]


# speedup vs baseline: 42.3190x; 1.4783x over previous
import functools

import jax
import jax.numpy as jnp
from jax import lax
from jax.experimental import pallas as pl
from jax.experimental.pallas import tpu as pltpu
from jax.experimental.pallas import tpu_sc as plsc

N_NODES = 10000
N_EDGES = 320000
D_IN = 128
D_HID = 64
N_CLS = 16

NC = 2
NS = 16
NW = NC * NS
CHUNK = 128
CHUNKS_PER_TILE = 80
E_PAD = NW * CHUNKS_PER_TILE * CHUNK
N_PAD = 10240
ROWS_PER_TILE = N_PAD // NS
DEGW = 16
NBUF = 4


def _sc_mesh():
    return plsc.VectorSubcoreMesh(
        core_axis_name="c", subcore_axis_name="s", num_cores=NC, num_subcores=NS
    )


def _degree_sc(dst_all):
    @functools.partial(
        pl.kernel,
        mesh=_sc_mesh(),
        out_type=jax.ShapeDtypeStruct((NC, N_PAD, DEGW), jnp.float32),
        scratch_types=[
            pltpu.VMEM((CHUNKS_PER_TILE, CHUNK), jnp.int32),
            pltpu.VMEM((CHUNK, DEGW), jnp.float32),
            pltpu.VMEM((ROWS_PER_TILE, DEGW), jnp.float32),
            pltpu.VMEM_SHARED((N_PAD, DEGW), jnp.float32),
            pltpu.SemaphoreType.DMA,
        ],
        compiler_params=pltpu.CompilerParams(use_tc_tiling_on_sc=False),
    )
    def deg_kernel(dst_hbm, out_hbm, idxv, onesv, slicev, acc_sh, sem):
        c = lax.axis_index("c")
        s = lax.axis_index("s")
        wid = s * NC + c

        def fill_ones(j, carry):
            onesv[j, :] = jnp.ones((DEGW,), jnp.float32)
            return carry

        lax.fori_loop(0, CHUNK, fill_ones, 0)

        def fill_zero(i, carry):
            slicev[i, :] = jnp.zeros((DEGW,), jnp.float32)
            return carry

        lax.fori_loop(0, ROWS_PER_TILE, fill_zero, 0)
        pltpu.sync_copy(slicev, acc_sh.at[pl.ds(s * ROWS_PER_TILE, ROWS_PER_TILE)])
        plsc.subcore_barrier()

        pltpu.sync_copy(dst_hbm.at[wid], idxv)

        win = 8

        def start(j, carry):
            pltpu.async_copy(onesv, acc_sh.at[idxv.at[j]], sem, add=True)
            return carry

        def slide(j, carry):
            pltpu.make_async_copy(onesv, acc_sh.at[idxv.at[j - win]], sem).wait()
            pltpu.async_copy(onesv, acc_sh.at[idxv.at[j]], sem, add=True)
            return carry

        def drain(j, carry):
            pltpu.make_async_copy(onesv, acc_sh.at[idxv.at[j]], sem).wait()
            return carry

        lax.fori_loop(0, win, start, 0)
        lax.fori_loop(win, CHUNKS_PER_TILE, slide, 0)
        lax.fori_loop(CHUNKS_PER_TILE - win, CHUNKS_PER_TILE, drain, 0)
        plsc.subcore_barrier()

        pltpu.sync_copy(acc_sh.at[pl.ds(s * ROWS_PER_TILE, ROWS_PER_TILE)], slicev)
        pltpu.sync_copy(
            slicev, out_hbm.at[c, pl.ds(s * ROWS_PER_TILE, ROWS_PER_TILE)]
        )

    return deg_kernel(dst_all)


def _aggregate_sc(y, src_all, dst_all, d):
    @functools.partial(
        pl.kernel,
        mesh=_sc_mesh(),
        out_type=jax.ShapeDtypeStruct((NC, N_PAD, d), jnp.float32),
        scratch_types=[
            pltpu.VMEM((CHUNKS_PER_TILE, CHUNK), jnp.int32),
            pltpu.VMEM((CHUNKS_PER_TILE, CHUNK), jnp.int32),
            pltpu.VMEM((NBUF, CHUNK, d), jnp.float32),
            pltpu.VMEM_SHARED((N_PAD, d), jnp.float32),
            [pltpu.SemaphoreType.DMA] * NBUF,
            [pltpu.SemaphoreType.DMA] * NBUF,
        ],
        compiler_params=pltpu.CompilerParams(use_tc_tiling_on_sc=False),
    )
    def agg_kernel(y_hbm, src_hbm, dst_hbm, out_hbm, srcv, dstv, rows, acc_sh, semg, sems):
        c = lax.axis_index("c")
        s = lax.axis_index("s")
        wid = s * NC + c
        pieces = ROWS_PER_TILE // CHUNK

        def fill_zero(i, carry):
            for k in range(d // 16):
                rows[0, i, pl.ds(k * 16, 16)] = jnp.zeros((16,), jnp.float32)
            return carry

        lax.fori_loop(0, CHUNK, fill_zero, 0)

        def zero_piece(p, carry):
            pltpu.sync_copy(
                rows.at[0], acc_sh.at[pl.ds(s * ROWS_PER_TILE + p * CHUNK, CHUNK)]
            )
            return carry

        lax.fori_loop(0, pieces, zero_piece, 0)
        plsc.subcore_barrier()

        pltpu.sync_copy(src_hbm.at[wid], srcv)
        pltpu.sync_copy(dst_hbm.at[wid], dstv)

        ngroups = CHUNKS_PER_TILE // NBUF

        def group(g, carry):
            base = g * NBUF
            gath = []
            for b in range(NBUF):
                @pl.when(g > 0)
                def _wait_prev(b=b, base=base):
                    pltpu.make_async_copy(
                        rows.at[b], acc_sh.at[dstv.at[base - NBUF + b]], sems[b]
                    ).wait()

                gath.append(
                    pltpu.async_copy(
                        y_hbm.at[srcv.at[base + b]], rows.at[b], semg[b]
                    )
                )
            for b in range(NBUF):
                gath[b].wait()
                pltpu.async_copy(
                    rows.at[b], acc_sh.at[dstv.at[base + b]], sems[b], add=True
                )
            return carry

        lax.fori_loop(0, ngroups, group, 0)
        last = (ngroups - 1) * NBUF
        for b in range(NBUF):
            pltpu.make_async_copy(
                rows.at[b], acc_sh.at[dstv.at[last + b]], sems[b]
            ).wait()
        plsc.subcore_barrier()

        def read_piece(p, carry):
            off = s * ROWS_PER_TILE + p * CHUNK
            pltpu.sync_copy(acc_sh.at[pl.ds(off, CHUNK)], rows.at[0])
            pltpu.sync_copy(rows.at[0], out_hbm.at[c, pl.ds(off, CHUNK)])
            return carry

        lax.fori_loop(0, pieces, read_piece, 0)

    return agg_kernel(y, src_all, dst_all)


_R = 512


def _dinv_block(degp):
    deg = degp[0, :, 0:1] + degp[1, :, 0:1] + 1.0
    return lax.rsqrt(deg)


def _mm1_body(x_ref, w_ref, degp_ref, y_ref):
    dinv = _dinv_block(degp_ref[...])
    xw = jnp.dot(x_ref[...], w_ref[...], preferred_element_type=jnp.float32)
    y_ref[...] = xw * dinv


def _mid_body(p_ref, y1_ref, b1_ref, w2_ref, degp_ref, y2_ref):
    dinv = _dinv_block(degp_ref[...])
    p = p_ref[...]
    h = jnp.maximum((p[0] + p[1] + y1_ref[...]) * dinv + b1_ref[...], 0.0)
    y2_ref[...] = (
        jnp.dot(h, w2_ref[...], preferred_element_type=jnp.float32) * dinv
    )


def _out_body(p_ref, y2_ref, b2_ref, degp_ref, o_ref):
    dinv = _dinv_block(degp_ref[...])
    p = p_ref[...]
    z = (p[0] + p[1] + y2_ref[...]) * dinv + b2_ref[...]
    m = jnp.max(z, axis=1, keepdims=True)
    lse = jnp.log(jnp.sum(jnp.exp(z - m), axis=1, keepdims=True)) + m
    o_ref[...] = z - lse


def _tc_mm1(x_pad, W1, degp):
    return pl.pallas_call(
        _mm1_body,
        grid=(N_PAD // _R,),
        in_specs=[
            pl.BlockSpec((_R, D_IN), lambda i: (i, 0)),
            pl.BlockSpec((D_IN, D_HID), lambda i: (0, 0)),
            pl.BlockSpec((NC, _R, DEGW), lambda i: (0, i, 0)),
        ],
        out_specs=pl.BlockSpec((_R, D_HID), lambda i: (i, 0)),
        out_shape=jax.ShapeDtypeStruct((N_PAD, D_HID), jnp.float32),
    )(x_pad, W1, degp)


def _tc_mid(p1, y1, b1, W2, degp):
    return pl.pallas_call(
        _mid_body,
        grid=(N_PAD // _R,),
        in_specs=[
            pl.BlockSpec((NC, _R, D_HID), lambda i: (0, i, 0)),
            pl.BlockSpec((_R, D_HID), lambda i: (i, 0)),
            pl.BlockSpec((1, D_HID), lambda i: (0, 0)),
            pl.BlockSpec((D_HID, N_CLS), lambda i: (0, 0)),
            pl.BlockSpec((NC, _R, DEGW), lambda i: (0, i, 0)),
        ],
        out_specs=pl.BlockSpec((_R, N_CLS), lambda i: (i, 0)),
        out_shape=jax.ShapeDtypeStruct((N_PAD, N_CLS), jnp.float32),
    )(p1, y1, b1.reshape(1, D_HID), W2, degp)


def _tc_out(p2, y2, b2, degp):
    return pl.pallas_call(
        _out_body,
        grid=(N_PAD // _R,),
        in_specs=[
            pl.BlockSpec((NC, _R, N_CLS), lambda i: (0, i, 0)),
            pl.BlockSpec((_R, N_CLS), lambda i: (i, 0)),
            pl.BlockSpec((1, N_CLS), lambda i: (0, 0)),
            pl.BlockSpec((NC, _R, DEGW), lambda i: (0, i, 0)),
        ],
        out_specs=pl.BlockSpec((_R, N_CLS), lambda i: (i, 0)),
        out_shape=jax.ShapeDtypeStruct((N_PAD, N_CLS), jnp.float32),
    )(p2, y2, b2.reshape(1, N_CLS), degp)


def kernel(x, edge_index, W1, b1, W2, b2):
    ei = edge_index.astype(jnp.int32)
    src, dst = ei[0], ei[1]
    n_extra = E_PAD - N_EDGES
    pad_ar = jnp.arange(n_extra, dtype=jnp.int32)
    src_all = jnp.concatenate([src, pad_ar % N_NODES]).reshape(
        NW, CHUNKS_PER_TILE, CHUNK
    )
    dst_all = jnp.concatenate(
        [dst, N_NODES + pad_ar % (N_PAD - N_NODES)]
    ).reshape(NW, CHUNKS_PER_TILE, CHUNK)
    x_pad = jnp.pad(x, ((0, N_PAD - N_NODES), (0, 0)))

    degp = _degree_sc(dst_all)
    y1 = _tc_mm1(x_pad, W1, degp)
    p1 = _aggregate_sc(y1, src_all, dst_all, D_HID)
    y2 = _tc_mid(p1, y1, b1, W2, degp)
    p2 = _aggregate_sc(y2, src_all, dst_all, N_CLS)
    out = _tc_out(p2, y2, b2, degp)
    return out[:N_NODES]


# speedup vs baseline: 47.7285x; 1.1278x over previous
import functools

import jax
import jax.numpy as jnp
from jax import lax
from jax.experimental import pallas as pl
from jax.experimental.pallas import tpu as pltpu
from jax.experimental.pallas import tpu_sc as plsc

N_NODES = 10000
N_EDGES = 320000
D_IN = 128
D_HID = 64
N_CLS = 16

NC = 2
NS = 16
NW = NC * NS
CHUNK = 128
CHUNKS_PER_TILE = 80
E_PAD = NW * CHUNKS_PER_TILE * CHUNK
N_PAD = 10240
ROWS_PER_TILE = N_PAD // NS
DEGW = 16
NBUF = 8


def _sc_mesh():
    return plsc.VectorSubcoreMesh(
        core_axis_name="c", subcore_axis_name="s", num_cores=NC, num_subcores=NS
    )


def _degree_sc(dst_all):
    @functools.partial(
        pl.kernel,
        mesh=_sc_mesh(),
        out_type=jax.ShapeDtypeStruct((NC, N_PAD, DEGW), jnp.float32),
        scratch_types=[
            pltpu.VMEM((CHUNKS_PER_TILE, CHUNK), jnp.int32),
            pltpu.VMEM((CHUNK, DEGW), jnp.float32),
            pltpu.VMEM((ROWS_PER_TILE, DEGW), jnp.float32),
            pltpu.VMEM_SHARED((N_PAD, DEGW), jnp.float32),
            pltpu.SemaphoreType.DMA,
        ],
        compiler_params=pltpu.CompilerParams(use_tc_tiling_on_sc=False),
    )
    def deg_kernel(dst_hbm, out_hbm, idxv, onesv, slicev, acc_sh, sem):
        c = lax.axis_index("c")
        s = lax.axis_index("s")
        wid = s * NC + c

        def fill_ones(j, carry):
            onesv[j, :] = jnp.ones((DEGW,), jnp.float32)
            return carry

        lax.fori_loop(0, CHUNK, fill_ones, 0)

        def fill_zero(i, carry):
            slicev[i, :] = jnp.zeros((DEGW,), jnp.float32)
            return carry

        lax.fori_loop(0, ROWS_PER_TILE, fill_zero, 0)
        pltpu.sync_copy(slicev, acc_sh.at[pl.ds(s * ROWS_PER_TILE, ROWS_PER_TILE)])
        plsc.subcore_barrier()

        pltpu.sync_copy(dst_hbm.at[wid], idxv)

        win = 8

        def start(j, carry):
            pltpu.async_copy(onesv, acc_sh.at[idxv.at[j]], sem, add=True)
            return carry

        def slide(j, carry):
            pltpu.make_async_copy(onesv, acc_sh.at[idxv.at[j - win]], sem).wait()
            pltpu.async_copy(onesv, acc_sh.at[idxv.at[j]], sem, add=True)
            return carry

        def drain(j, carry):
            pltpu.make_async_copy(onesv, acc_sh.at[idxv.at[j]], sem).wait()
            return carry

        lax.fori_loop(0, win, start, 0)
        lax.fori_loop(win, CHUNKS_PER_TILE, slide, 0)
        lax.fori_loop(CHUNKS_PER_TILE - win, CHUNKS_PER_TILE, drain, 0)
        plsc.subcore_barrier()

        pltpu.sync_copy(acc_sh.at[pl.ds(s * ROWS_PER_TILE, ROWS_PER_TILE)], slicev)
        pltpu.sync_copy(
            slicev, out_hbm.at[c, pl.ds(s * ROWS_PER_TILE, ROWS_PER_TILE)]
        )

    return deg_kernel(dst_all)


def _aggregate_sc(y, src_all, dst_all, d):
    @functools.partial(
        pl.kernel,
        mesh=_sc_mesh(),
        out_type=jax.ShapeDtypeStruct((NC, N_PAD, d), jnp.float32),
        scratch_types=[
            pltpu.VMEM((CHUNKS_PER_TILE, CHUNK), jnp.int32),
            pltpu.VMEM((CHUNKS_PER_TILE, CHUNK), jnp.int32),
            pltpu.VMEM((NBUF, CHUNK, d), jnp.float32),
            pltpu.VMEM_SHARED((N_PAD, d), jnp.float32),
            [pltpu.SemaphoreType.DMA] * NBUF,
            [pltpu.SemaphoreType.DMA] * NBUF,
        ],
        compiler_params=pltpu.CompilerParams(use_tc_tiling_on_sc=False),
    )
    def agg_kernel(y_hbm, src_hbm, dst_hbm, out_hbm, srcv, dstv, rows, acc_sh, semg, sems):
        c = lax.axis_index("c")
        s = lax.axis_index("s")
        wid = s * NC + c
        pieces = ROWS_PER_TILE // CHUNK

        def fill_zero(i, carry):
            for k in range(d // 16):
                rows[0, i, pl.ds(k * 16, 16)] = jnp.zeros((16,), jnp.float32)
            return carry

        lax.fori_loop(0, CHUNK, fill_zero, 0)

        def zero_piece(p, carry):
            pltpu.sync_copy(
                rows.at[0], acc_sh.at[pl.ds(s * ROWS_PER_TILE + p * CHUNK, CHUNK)]
            )
            return carry

        lax.fori_loop(0, pieces, zero_piece, 0)
        plsc.subcore_barrier()

        pltpu.sync_copy(src_hbm.at[wid], srcv)
        pltpu.sync_copy(dst_hbm.at[wid], dstv)

        ngroups = CHUNKS_PER_TILE // NBUF

        def group(g, carry):
            base = g * NBUF
            gath = []
            for b in range(NBUF):
                @pl.when(g > 0)
                def _wait_prev(b=b, base=base):
                    pltpu.make_async_copy(
                        rows.at[b], acc_sh.at[dstv.at[base - NBUF + b]], sems[b]
                    ).wait()

                gath.append(
                    pltpu.async_copy(
                        y_hbm.at[srcv.at[base + b]], rows.at[b], semg[b]
                    )
                )
            for b in range(NBUF):
                gath[b].wait()
                pltpu.async_copy(
                    rows.at[b], acc_sh.at[dstv.at[base + b]], sems[b], add=True
                )
            return carry

        lax.fori_loop(0, ngroups, group, 0)
        last = (ngroups - 1) * NBUF
        for b in range(NBUF):
            pltpu.make_async_copy(
                rows.at[b], acc_sh.at[dstv.at[last + b]], sems[b]
            ).wait()
        plsc.subcore_barrier()

        def read_piece(p, carry):
            off = s * ROWS_PER_TILE + p * CHUNK
            pltpu.sync_copy(acc_sh.at[pl.ds(off, CHUNK)], rows.at[0])
            pltpu.sync_copy(rows.at[0], out_hbm.at[c, pl.ds(off, CHUNK)])
            return carry

        lax.fori_loop(0, pieces, read_piece, 0)

    return agg_kernel(y, src_all, dst_all)


_R = 1024


def _dinv_block(degp):
    deg = degp[0, :, 0:1] + degp[1, :, 0:1] + 1.0
    return lax.rsqrt(deg)


def _mm1_body(x_ref, w_ref, degp_ref, y_ref):
    dinv = _dinv_block(degp_ref[...])
    xw = jnp.dot(x_ref[...], w_ref[...], preferred_element_type=jnp.float32)
    y_ref[...] = xw * dinv


def _mid_body(p_ref, y1_ref, b1_ref, w2_ref, degp_ref, y2_ref):
    dinv = _dinv_block(degp_ref[...])
    p = p_ref[...]
    h = jnp.maximum((p[0] + p[1] + y1_ref[...]) * dinv + b1_ref[...], 0.0)
    y2_ref[...] = (
        jnp.dot(h, w2_ref[...], preferred_element_type=jnp.float32) * dinv
    )


def _out_body(p_ref, y2_ref, b2_ref, degp_ref, o_ref):
    dinv = _dinv_block(degp_ref[...])
    p = p_ref[...]
    z = (p[0] + p[1] + y2_ref[...]) * dinv + b2_ref[...]
    m = jnp.max(z, axis=1, keepdims=True)
    lse = jnp.log(jnp.sum(jnp.exp(z - m), axis=1, keepdims=True)) + m
    o_ref[...] = z - lse


def _tc_mm1(x_pad, W1, degp):
    return pl.pallas_call(
        _mm1_body,
        grid=(N_PAD // _R,),
        in_specs=[
            pl.BlockSpec((_R, D_IN), lambda i: (i, 0)),
            pl.BlockSpec((D_IN, D_HID), lambda i: (0, 0)),
            pl.BlockSpec((NC, _R, DEGW), lambda i: (0, i, 0)),
        ],
        out_specs=pl.BlockSpec((_R, D_HID), lambda i: (i, 0)),
        out_shape=jax.ShapeDtypeStruct((N_PAD, D_HID), jnp.float32),
    )(x_pad, W1, degp)


def _tc_mid(p1, y1, b1, W2, degp):
    return pl.pallas_call(
        _mid_body,
        grid=(N_PAD // _R,),
        in_specs=[
            pl.BlockSpec((NC, _R, D_HID), lambda i: (0, i, 0)),
            pl.BlockSpec((_R, D_HID), lambda i: (i, 0)),
            pl.BlockSpec((1, D_HID), lambda i: (0, 0)),
            pl.BlockSpec((D_HID, N_CLS), lambda i: (0, 0)),
            pl.BlockSpec((NC, _R, DEGW), lambda i: (0, i, 0)),
        ],
        out_specs=pl.BlockSpec((_R, N_CLS), lambda i: (i, 0)),
        out_shape=jax.ShapeDtypeStruct((N_PAD, N_CLS), jnp.float32),
    )(p1, y1, b1.reshape(1, D_HID), W2, degp)


def _tc_out(p2, y2, b2, degp):
    return pl.pallas_call(
        _out_body,
        grid=(N_PAD // _R,),
        in_specs=[
            pl.BlockSpec((NC, _R, N_CLS), lambda i: (0, i, 0)),
            pl.BlockSpec((_R, N_CLS), lambda i: (i, 0)),
            pl.BlockSpec((1, N_CLS), lambda i: (0, 0)),
            pl.BlockSpec((NC, _R, DEGW), lambda i: (0, i, 0)),
        ],
        out_specs=pl.BlockSpec((_R, N_CLS), lambda i: (i, 0)),
        out_shape=jax.ShapeDtypeStruct((N_PAD, N_CLS), jnp.float32),
    )(p2, y2, b2.reshape(1, N_CLS), degp)


def kernel(x, edge_index, W1, b1, W2, b2):
    ei = edge_index.astype(jnp.int32)
    src, dst = ei[0], ei[1]
    n_extra = E_PAD - N_EDGES
    pad_ar = jnp.arange(n_extra, dtype=jnp.int32)
    src_all = jnp.concatenate([src, pad_ar % N_NODES]).reshape(
        NW, CHUNKS_PER_TILE, CHUNK
    )
    dst_all = jnp.concatenate(
        [dst, N_NODES + pad_ar % (N_PAD - N_NODES)]
    ).reshape(NW, CHUNKS_PER_TILE, CHUNK)
    x_pad = jnp.pad(x, ((0, N_PAD - N_NODES), (0, 0)))

    degp = _degree_sc(dst_all)
    y1 = _tc_mm1(x_pad, W1, degp)
    p1 = _aggregate_sc(y1, src_all, dst_all, D_HID)
    y2 = _tc_mid(p1, y1, b1, W2, degp)
    p2 = _aggregate_sc(y2, src_all, dst_all, N_CLS)
    out = _tc_out(p2, y2, b2, degp)
    return out[:N_NODES]


# speedup vs baseline: 49.2977x; 1.0329x over previous
import functools

import jax
import jax.numpy as jnp
from jax import lax
from jax.experimental import pallas as pl
from jax.experimental.pallas import tpu as pltpu
from jax.experimental.pallas import tpu_sc as plsc

N_NODES = 10000
N_EDGES = 320000
D_IN = 128
D_HID = 64
N_CLS = 16

NC = 2
NS = 16
NW = NC * NS
CHUNK = 128
CHUNKS_PER_TILE = 80
E_PAD = NW * CHUNKS_PER_TILE * CHUNK
N_PAD = 10240
ROWS_PER_TILE = N_PAD // NS
DEGW = 16
NBUF = 8
NODES_PER_TILE = N_PAD // NW


def _sc_mesh():
    return plsc.VectorSubcoreMesh(
        core_axis_name="c", subcore_axis_name="s", num_cores=NC, num_subcores=NS
    )


def _newton_rsqrt(d):
    scale = jnp.ones_like(d)
    m = d
    for _ in range(10):
        big = m > 2.0
        m = jnp.where(big, m * 0.25, m)
        scale = jnp.where(big, scale * 0.5, scale)
    y = jnp.ones_like(m)
    for _ in range(5):
        y = y * (1.5 - 0.5 * m * y * y)
    return y * scale


def _dinv_sc(dst_all):
    nchunks = 2 * CHUNKS_PER_TILE

    @functools.partial(
        pl.kernel,
        mesh=_sc_mesh(),
        out_type=[
            jax.ShapeDtypeStruct((N_PAD // 2, 128), jnp.float32),
            jax.ShapeDtypeStruct((N_PAD, DEGW), jnp.float32),
        ],
        scratch_types=[
            pltpu.VMEM((nchunks, CHUNK), jnp.int32),
            pltpu.VMEM((CHUNK, DEGW), jnp.float32),
            pltpu.VMEM((ROWS_PER_TILE, DEGW), jnp.float32),
            pltpu.VMEM((NODES_PER_TILE, DEGW), jnp.float32),
            pltpu.VMEM((NODES_PER_TILE // 2, 128), jnp.float32),
            pltpu.VMEM_SHARED((N_PAD, DEGW), jnp.float32),
            pltpu.SemaphoreType.DMA,
        ],
        compiler_params=pltpu.CompilerParams(use_tc_tiling_on_sc=False),
    )
    def deg_kernel(dst_hbm, d64_hbm, d16_hbm, idxv, onesv, slicev, d16v, d64v,
                   acc_sh, sem):
        c = lax.axis_index("c")
        s = lax.axis_index("s")

        def fill_ones(j, carry):
            onesv[j, :] = jnp.ones((DEGW,), jnp.float32)
            return carry

        lax.fori_loop(0, CHUNK, fill_ones, 0)

        def fill_zero(i, carry):
            slicev[i, :] = jnp.zeros((DEGW,), jnp.float32)
            return carry

        lax.fori_loop(0, ROWS_PER_TILE, fill_zero, 0)
        pltpu.sync_copy(slicev, acc_sh.at[pl.ds(s * ROWS_PER_TILE, ROWS_PER_TILE)])
        plsc.subcore_barrier()

        pltpu.sync_copy(dst_hbm.at[2 * s], idxv.at[pl.ds(0, CHUNKS_PER_TILE)])
        pltpu.sync_copy(
            dst_hbm.at[2 * s + 1], idxv.at[pl.ds(CHUNKS_PER_TILE, CHUNKS_PER_TILE)]
        )

        win = 8

        def start(j, carry):
            pltpu.async_copy(onesv, acc_sh.at[idxv.at[j]], sem, add=True)
            return carry

        def slide(j, carry):
            pltpu.make_async_copy(onesv, acc_sh.at[idxv.at[j - win]], sem).wait()
            pltpu.async_copy(onesv, acc_sh.at[idxv.at[j]], sem, add=True)
            return carry

        def drain(j, carry):
            pltpu.make_async_copy(onesv, acc_sh.at[idxv.at[j]], sem).wait()
            return carry

        lax.fori_loop(0, win, start, 0)
        lax.fori_loop(win, nchunks, slide, 0)
        lax.fori_loop(nchunks - win, nchunks, drain, 0)
        plsc.subcore_barrier()

        base_n = (c * NS + s) * NODES_PER_TILE
        base_h = (c * NS + s) * (NODES_PER_TILE // 2)
        pltpu.sync_copy(acc_sh.at[pl.ds(base_n, NODES_PER_TILE)], d16v)

        def dinv_node(i, carry):
            d = d16v[i, :] + 1.0
            y = _newton_rsqrt(d)
            d16v[i, :] = y
            r = i // 2
            h = (i % 2) * 64
            for k in range(4):
                d64v[r, pl.ds(h + k * 16, 16)] = y
            return carry

        lax.fori_loop(0, NODES_PER_TILE, dinv_node, 0)
        pltpu.sync_copy(d16v, d16_hbm.at[pl.ds(base_n, NODES_PER_TILE)])
        pltpu.sync_copy(d64v, d64_hbm.at[pl.ds(base_h, NODES_PER_TILE // 2)])

    return deg_kernel(dst_all)


def _aggregate_sc(y, src_all, dst_all, d):
    @functools.partial(
        pl.kernel,
        mesh=_sc_mesh(),
        out_type=jax.ShapeDtypeStruct((NC, N_PAD, d), jnp.float32),
        scratch_types=[
            pltpu.VMEM((CHUNKS_PER_TILE, CHUNK), jnp.int32),
            pltpu.VMEM((CHUNKS_PER_TILE, CHUNK), jnp.int32),
            pltpu.VMEM((NBUF, CHUNK, d), jnp.float32),
            pltpu.VMEM_SHARED((N_PAD, d), jnp.float32),
            [pltpu.SemaphoreType.DMA] * NBUF,
            [pltpu.SemaphoreType.DMA] * NBUF,
        ],
        compiler_params=pltpu.CompilerParams(use_tc_tiling_on_sc=False),
    )
    def agg_kernel(y_hbm, src_hbm, dst_hbm, out_hbm, srcv, dstv, rows, acc_sh, semg, sems):
        c = lax.axis_index("c")
        s = lax.axis_index("s")
        wid = s * NC + c
        pieces = ROWS_PER_TILE // CHUNK

        def fill_zero(i, carry):
            for k in range(d // 16):
                rows[0, i, pl.ds(k * 16, 16)] = jnp.zeros((16,), jnp.float32)
            return carry

        lax.fori_loop(0, CHUNK, fill_zero, 0)

        def zero_piece(p, carry):
            pltpu.sync_copy(
                rows.at[0], acc_sh.at[pl.ds(s * ROWS_PER_TILE + p * CHUNK, CHUNK)]
            )
            return carry

        lax.fori_loop(0, pieces, zero_piece, 0)
        plsc.subcore_barrier()

        pltpu.sync_copy(src_hbm.at[wid], srcv)
        pltpu.sync_copy(dst_hbm.at[wid], dstv)

        ngroups = CHUNKS_PER_TILE // NBUF

        def group(g, carry):
            base = g * NBUF
            gath = []
            for b in range(NBUF):
                @pl.when(g > 0)
                def _wait_prev(b=b, base=base):
                    pltpu.make_async_copy(
                        rows.at[b], acc_sh.at[dstv.at[base - NBUF + b]], sems[b]
                    ).wait()

                gath.append(
                    pltpu.async_copy(
                        y_hbm.at[srcv.at[base + b]], rows.at[b], semg[b]
                    )
                )
            for b in range(NBUF):
                gath[b].wait()
                pltpu.async_copy(
                    rows.at[b], acc_sh.at[dstv.at[base + b]], sems[b], add=True
                )
            return carry

        lax.fori_loop(0, ngroups, group, 0)
        last = (ngroups - 1) * NBUF
        for b in range(NBUF):
            pltpu.make_async_copy(
                rows.at[b], acc_sh.at[dstv.at[last + b]], sems[b]
            ).wait()
        plsc.subcore_barrier()

        def read_piece(p, carry):
            off = s * ROWS_PER_TILE + p * CHUNK
            pltpu.sync_copy(acc_sh.at[pl.ds(off, CHUNK)], rows.at[0])
            pltpu.sync_copy(rows.at[0], out_hbm.at[c, pl.ds(off, CHUNK)])
            return carry

        lax.fori_loop(0, pieces, read_piece, 0)

    return agg_kernel(y, src_all, dst_all)


_R = 512


def _mm1_body(x_ref, w_ref, d64_ref, y_ref):
    y_ref[...] = (
        jnp.dot(x_ref[...], w_ref[...], preferred_element_type=jnp.float32)
        * d64_ref[...]
    )


def _mid_body(p_ref, y1_ref, b1_ref, w2_ref, d64_ref, y2_ref):
    p = p_ref[...]
    d64 = d64_ref[...]
    h = jnp.maximum((p[0] + p[1] + y1_ref[...]) * d64 + b1_ref[...], 0.0)
    y2_ref[...] = jnp.dot(
        h * d64, w2_ref[...], preferred_element_type=jnp.float32
    )


def _out_body(p_ref, y2_ref, b2_ref, d16_ref, o_ref):
    p = p_ref[...]
    z = (p[0] + p[1] + y2_ref[...]) * d16_ref[...] + b2_ref[...]
    m = jnp.max(z, axis=1, keepdims=True)
    lse = jnp.log(jnp.sum(jnp.exp(z - m), axis=1, keepdims=True)) + m
    o_ref[...] = z - lse


def _tc_mm1(x_packed, Wbd1, d64p):
    n = N_PAD // 2
    return pl.pallas_call(
        _mm1_body,
        grid=(n // _R,),
        in_specs=[
            pl.BlockSpec((_R, 2 * D_IN), lambda i: (i, 0)),
            pl.BlockSpec((2 * D_IN, 2 * D_HID), lambda i: (0, 0)),
            pl.BlockSpec((_R, 128), lambda i: (i, 0)),
        ],
        out_specs=pl.BlockSpec((_R, 128), lambda i: (i, 0)),
        out_shape=jax.ShapeDtypeStruct((n, 128), jnp.float32),
    )(x_packed, Wbd1, d64p)


def _tc_mid(p1p, y1p, b1p, Wbd2, d64p):
    n = N_PAD // 2
    return pl.pallas_call(
        _mid_body,
        grid=(n // _R,),
        in_specs=[
            pl.BlockSpec((NC, _R, 128), lambda i: (0, i, 0)),
            pl.BlockSpec((_R, 128), lambda i: (i, 0)),
            pl.BlockSpec((1, 128), lambda i: (0, 0)),
            pl.BlockSpec((128, 2 * N_CLS), lambda i: (0, 0)),
            pl.BlockSpec((_R, 128), lambda i: (i, 0)),
        ],
        out_specs=pl.BlockSpec((_R, 2 * N_CLS), lambda i: (i, 0)),
        out_shape=jax.ShapeDtypeStruct((n, 2 * N_CLS), jnp.float32),
    )(p1p, y1p, b1p, Wbd2, d64p)


_R3 = 1024


def _tc_out(p2, y2_16, b2, d16):
    return pl.pallas_call(
        _out_body,
        grid=(N_PAD // _R3,),
        in_specs=[
            pl.BlockSpec((NC, _R3, N_CLS), lambda i: (0, i, 0)),
            pl.BlockSpec((_R3, N_CLS), lambda i: (i, 0)),
            pl.BlockSpec((1, N_CLS), lambda i: (0, 0)),
            pl.BlockSpec((_R3, N_CLS), lambda i: (i, 0)),
        ],
        out_specs=pl.BlockSpec((_R3, N_CLS), lambda i: (i, 0)),
        out_shape=jax.ShapeDtypeStruct((N_PAD, N_CLS), jnp.float32),
    )(p2, y2_16, b2.reshape(1, N_CLS), d16)


def _blockdiag2(W):
    k, m = W.shape
    Z = jnp.zeros((k, m), W.dtype)
    return jnp.concatenate(
        [jnp.concatenate([W, Z], axis=1), jnp.concatenate([Z, W], axis=1)], axis=0
    )


def kernel(x, edge_index, W1, b1, W2, b2):
    ei = edge_index.astype(jnp.int32)
    src, dst = ei[0], ei[1]
    n_extra = E_PAD - N_EDGES
    pad_ar = jnp.arange(n_extra, dtype=jnp.int32)
    src_all = jnp.concatenate([src, pad_ar % N_NODES]).reshape(
        NW, CHUNKS_PER_TILE, CHUNK
    )
    dst_all = jnp.concatenate(
        [dst, N_NODES + pad_ar % (N_PAD - N_NODES)]
    ).reshape(NW, CHUNKS_PER_TILE, CHUNK)
    x_packed = jnp.pad(x, ((0, N_PAD - N_NODES), (0, 0))).reshape(
        N_PAD // 2, 2 * D_IN
    )

    d64p, d16 = _dinv_sc(dst_all)
    y1p = _tc_mm1(x_packed, _blockdiag2(W1), d64p)
    p1 = _aggregate_sc(y1p.reshape(N_PAD, D_HID), src_all, dst_all, D_HID)
    y2p = _tc_mid(
        p1.reshape(NC, N_PAD // 2, 128),
        y1p,
        jnp.concatenate([b1, b1]).reshape(1, 128),
        _blockdiag2(W2),
        d64p,
    )
    p2 = _aggregate_sc(y2p.reshape(N_PAD, N_CLS), src_all, dst_all, N_CLS)
    out = _tc_out(p2, y2p.reshape(N_PAD, N_CLS), b2, d16)
    return out[:N_NODES]


# speedup vs baseline: 52.2034x; 1.0589x over previous
import functools

import jax
import jax.numpy as jnp
from jax import lax
from jax.experimental import pallas as pl
from jax.experimental.pallas import tpu as pltpu
from jax.experimental.pallas import tpu_sc as plsc

N_NODES = 10000
N_EDGES = 320000
D_IN = 128
D_HID = 64
N_CLS = 16

NC = 2
NS = 16
NW = NC * NS
CHUNK = 128
CHUNKS_PER_TILE = 80
E_PAD = NW * CHUNKS_PER_TILE * CHUNK
N_PAD = 10240
ROWS_PER_TILE = N_PAD // NS
DEGW = 16
NBUF = 8
NODES_PER_TILE = N_PAD // NW


def _sc_mesh():
    return plsc.VectorSubcoreMesh(
        core_axis_name="c", subcore_axis_name="s", num_cores=NC, num_subcores=NS
    )


def _newton_rsqrt(d):
    scale = jnp.ones_like(d)
    m = d
    for _ in range(10):
        big = m > 2.0
        m = jnp.where(big, m * 0.25, m)
        scale = jnp.where(big, scale * 0.5, scale)
    y = jnp.ones_like(m)
    for _ in range(5):
        y = y * (1.5 - 0.5 * m * y * y)
    return y * scale


def _dinv_sc(dst_all):
    nchunks = 2 * CHUNKS_PER_TILE

    @functools.partial(
        pl.kernel,
        mesh=_sc_mesh(),
        out_type=[
            jax.ShapeDtypeStruct((N_PAD // 2, 128), jnp.float32),
            jax.ShapeDtypeStruct((N_PAD, DEGW), jnp.float32),
        ],
        scratch_types=[
            pltpu.VMEM((nchunks, CHUNK), jnp.int32),
            pltpu.VMEM((CHUNK, DEGW), jnp.float32),
            pltpu.VMEM((ROWS_PER_TILE, DEGW), jnp.float32),
            pltpu.VMEM((NODES_PER_TILE, DEGW), jnp.float32),
            pltpu.VMEM((NODES_PER_TILE // 2, 128), jnp.float32),
            pltpu.VMEM((NODES_PER_TILE,), jnp.float32),
            pltpu.VMEM_SHARED((N_PAD, DEGW), jnp.float32),
            pltpu.SemaphoreType.DMA,
        ],
        compiler_params=pltpu.CompilerParams(
            use_tc_tiling_on_sc=False, needs_layout_passes=False
        ),
    )
    def deg_kernel(dst_hbm, d64_hbm, d16_hbm, idxv, onesv, slicev, d16v, d64v,
                   dflat, acc_sh, sem):
        c = lax.axis_index("c")
        s = lax.axis_index("s")

        def fill_ones(j, carry):
            onesv[j, :] = jnp.ones((DEGW,), jnp.float32)
            return carry

        lax.fori_loop(0, CHUNK, fill_ones, 0)

        def fill_zero(i, carry):
            slicev[i, :] = jnp.zeros((DEGW,), jnp.float32)
            return carry

        lax.fori_loop(0, ROWS_PER_TILE, fill_zero, 0)
        pltpu.sync_copy(slicev, acc_sh.at[pl.ds(s * ROWS_PER_TILE, ROWS_PER_TILE)])
        plsc.subcore_barrier()

        pltpu.sync_copy(dst_hbm.at[2 * s], idxv.at[pl.ds(0, CHUNKS_PER_TILE)])
        pltpu.sync_copy(
            dst_hbm.at[2 * s + 1], idxv.at[pl.ds(CHUNKS_PER_TILE, CHUNKS_PER_TILE)]
        )

        win = 8

        def start(j, carry):
            pltpu.async_copy(onesv, acc_sh.at[idxv.at[j]], sem, add=True)
            return carry

        def slide(j, carry):
            pltpu.make_async_copy(onesv, acc_sh.at[idxv.at[j - win]], sem).wait()
            pltpu.async_copy(onesv, acc_sh.at[idxv.at[j]], sem, add=True)
            return carry

        def drain(j, carry):
            pltpu.make_async_copy(onesv, acc_sh.at[idxv.at[j]], sem).wait()
            return carry

        lax.fori_loop(0, win, start, 0)
        lax.fori_loop(win, nchunks, slide, 0)
        lax.fori_loop(nchunks - win, nchunks, drain, 0)
        plsc.subcore_barrier()

        base_n = (c * NS + s) * NODES_PER_TILE
        base_h = (c * NS + s) * (NODES_PER_TILE // 2)
        pltpu.sync_copy(acc_sh.at[pl.ds(base_n, NODES_PER_TILE)], d16v)

        iota16 = lax.iota(jnp.int32, 16)
        zero16 = jnp.zeros((16,), jnp.int32)

        def dinv_vec(v, carry):
            dg = plsc.load_gather(d16v, [v * 16 + iota16, zero16])
            y = _newton_rsqrt(dg + 1.0)
            dflat[pl.ds(v * 16, 16)] = y
            return carry

        lax.fori_loop(0, NODES_PER_TILE // 16, dinv_vec, 0)

        def dinv_node(i, carry):
            bv = plsc.load_gather(dflat, [jnp.full((16,), i, jnp.int32)])
            d16v[i, :] = bv
            r = i // 2
            h = (i % 2) * 64
            for k in range(4):
                d64v[r, pl.ds(h + k * 16, 16)] = bv
            return carry

        lax.fori_loop(0, NODES_PER_TILE, dinv_node, 0)
        pltpu.sync_copy(d16v, d16_hbm.at[pl.ds(base_n, NODES_PER_TILE)])
        pltpu.sync_copy(d64v, d64_hbm.at[pl.ds(base_h, NODES_PER_TILE // 2)])

    return deg_kernel(dst_all)


def _aggregate_sc(y, src_all, dst_all, d):
    @functools.partial(
        pl.kernel,
        mesh=_sc_mesh(),
        out_type=jax.ShapeDtypeStruct((NC, N_PAD, d), jnp.float32),
        scratch_types=[
            pltpu.VMEM((CHUNKS_PER_TILE, CHUNK), jnp.int32),
            pltpu.VMEM((CHUNKS_PER_TILE, CHUNK), jnp.int32),
            pltpu.VMEM((NBUF, CHUNK, d), jnp.float32),
            pltpu.VMEM_SHARED((N_PAD, d), jnp.float32),
            [pltpu.SemaphoreType.DMA] * NBUF,
            [pltpu.SemaphoreType.DMA] * NBUF,
        ],
        compiler_params=pltpu.CompilerParams(use_tc_tiling_on_sc=False),
    )
    def agg_kernel(y_hbm, src_hbm, dst_hbm, out_hbm, srcv, dstv, rows, acc_sh, semg, sems):
        c = lax.axis_index("c")
        s = lax.axis_index("s")
        wid = s * NC + c
        pieces = ROWS_PER_TILE // CHUNK

        def fill_zero(i, carry):
            for k in range(d // 16):
                rows[0, i, pl.ds(k * 16, 16)] = jnp.zeros((16,), jnp.float32)
            return carry

        lax.fori_loop(0, CHUNK, fill_zero, 0)

        def zero_piece(p, carry):
            pltpu.sync_copy(
                rows.at[0], acc_sh.at[pl.ds(s * ROWS_PER_TILE + p * CHUNK, CHUNK)]
            )
            return carry

        lax.fori_loop(0, pieces, zero_piece, 0)
        plsc.subcore_barrier()

        pltpu.sync_copy(src_hbm.at[wid], srcv)
        pltpu.sync_copy(dst_hbm.at[wid], dstv)

        ngroups = CHUNKS_PER_TILE // NBUF

        def group(g, carry):
            base = g * NBUF
            gath = []
            for b in range(NBUF):
                @pl.when(g > 0)
                def _wait_prev(b=b, base=base):
                    pltpu.make_async_copy(
                        rows.at[b], acc_sh.at[dstv.at[base - NBUF + b]], sems[b]
                    ).wait()

                gath.append(
                    pltpu.async_copy(
                        y_hbm.at[srcv.at[base + b]], rows.at[b], semg[b]
                    )
                )
            for b in range(NBUF):
                gath[b].wait()
                pltpu.async_copy(
                    rows.at[b], acc_sh.at[dstv.at[base + b]], sems[b], add=True
                )
            return carry

        lax.fori_loop(0, ngroups, group, 0)
        last = (ngroups - 1) * NBUF
        for b in range(NBUF):
            pltpu.make_async_copy(
                rows.at[b], acc_sh.at[dstv.at[last + b]], sems[b]
            ).wait()
        plsc.subcore_barrier()

        def read_piece(p, carry):
            off = s * ROWS_PER_TILE + p * CHUNK
            pltpu.sync_copy(acc_sh.at[pl.ds(off, CHUNK)], rows.at[0])
            pltpu.sync_copy(rows.at[0], out_hbm.at[c, pl.ds(off, CHUNK)])
            return carry

        lax.fori_loop(0, pieces, read_piece, 0)

    return agg_kernel(y, src_all, dst_all)


_R = 512


def _mm1_body(x_ref, w_ref, d64_ref, y_ref):
    y_ref[...] = (
        jnp.dot(x_ref[...], w_ref[...], preferred_element_type=jnp.float32)
        * d64_ref[...]
    )


def _mid_body(p_ref, y1_ref, b1_ref, w2_ref, d64_ref, y2_ref):
    p = p_ref[...]
    d64 = d64_ref[...]
    h = jnp.maximum((p[0] + p[1] + y1_ref[...]) * d64 + b1_ref[...], 0.0)
    y2_ref[...] = jnp.dot(
        h * d64, w2_ref[...], preferred_element_type=jnp.float32
    )


def _out_body(p_ref, y2_ref, b2_ref, d16_ref, o_ref):
    p = p_ref[...]
    z = (p[0] + p[1] + y2_ref[...]) * d16_ref[...] + b2_ref[...]
    m = jnp.max(z, axis=1, keepdims=True)
    lse = jnp.log(jnp.sum(jnp.exp(z - m), axis=1, keepdims=True)) + m
    o_ref[...] = z - lse


def _tc_mm1(x_packed, Wbd1, d64p):
    n = N_PAD // 2
    return pl.pallas_call(
        _mm1_body,
        grid=(n // _R,),
        in_specs=[
            pl.BlockSpec((_R, 2 * D_IN), lambda i: (i, 0)),
            pl.BlockSpec((2 * D_IN, 2 * D_HID), lambda i: (0, 0)),
            pl.BlockSpec((_R, 128), lambda i: (i, 0)),
        ],
        out_specs=pl.BlockSpec((_R, 128), lambda i: (i, 0)),
        out_shape=jax.ShapeDtypeStruct((n, 128), jnp.float32),
    )(x_packed, Wbd1, d64p)


def _tc_mid(p1p, y1p, b1p, Wbd2, d64p):
    n = N_PAD // 2
    return pl.pallas_call(
        _mid_body,
        grid=(n // _R,),
        in_specs=[
            pl.BlockSpec((NC, _R, 128), lambda i: (0, i, 0)),
            pl.BlockSpec((_R, 128), lambda i: (i, 0)),
            pl.BlockSpec((1, 128), lambda i: (0, 0)),
            pl.BlockSpec((128, 2 * N_CLS), lambda i: (0, 0)),
            pl.BlockSpec((_R, 128), lambda i: (i, 0)),
        ],
        out_specs=pl.BlockSpec((_R, 2 * N_CLS), lambda i: (i, 0)),
        out_shape=jax.ShapeDtypeStruct((n, 2 * N_CLS), jnp.float32),
    )(p1p, y1p, b1p, Wbd2, d64p)


_R3 = 1024


def _tc_out(p2, y2_16, b2, d16):
    return pl.pallas_call(
        _out_body,
        grid=(N_PAD // _R3,),
        in_specs=[
            pl.BlockSpec((NC, _R3, N_CLS), lambda i: (0, i, 0)),
            pl.BlockSpec((_R3, N_CLS), lambda i: (i, 0)),
            pl.BlockSpec((1, N_CLS), lambda i: (0, 0)),
            pl.BlockSpec((_R3, N_CLS), lambda i: (i, 0)),
        ],
        out_specs=pl.BlockSpec((_R3, N_CLS), lambda i: (i, 0)),
        out_shape=jax.ShapeDtypeStruct((N_PAD, N_CLS), jnp.float32),
    )(p2, y2_16, b2.reshape(1, N_CLS), d16)


def _blockdiag2(W):
    k, m = W.shape
    Z = jnp.zeros((k, m), W.dtype)
    return jnp.concatenate(
        [jnp.concatenate([W, Z], axis=1), jnp.concatenate([Z, W], axis=1)], axis=0
    )


def kernel(x, edge_index, W1, b1, W2, b2):
    ei = edge_index.astype(jnp.int32)
    src, dst = ei[0], ei[1]
    n_extra = E_PAD - N_EDGES
    pad_ar = jnp.arange(n_extra, dtype=jnp.int32)
    src_all = jnp.concatenate([src, pad_ar % N_NODES]).reshape(
        NW, CHUNKS_PER_TILE, CHUNK
    )
    dst_all = jnp.concatenate(
        [dst, N_NODES + pad_ar % (N_PAD - N_NODES)]
    ).reshape(NW, CHUNKS_PER_TILE, CHUNK)
    x_packed = jnp.pad(x, ((0, N_PAD - N_NODES), (0, 0))).reshape(
        N_PAD // 2, 2 * D_IN
    )

    d64p, d16 = _dinv_sc(dst_all)
    y1p = _tc_mm1(x_packed, _blockdiag2(W1), d64p)
    p1 = _aggregate_sc(y1p.reshape(N_PAD, D_HID), src_all, dst_all, D_HID)
    y2p = _tc_mid(
        p1.reshape(NC, N_PAD // 2, 128),
        y1p,
        jnp.concatenate([b1, b1]).reshape(1, 128),
        _blockdiag2(W2),
        d64p,
    )
    p2 = _aggregate_sc(y2p.reshape(N_PAD, N_CLS), src_all, dst_all, N_CLS)
    out = _tc_out(p2, y2p.reshape(N_PAD, N_CLS), b2, d16)
    return out[:N_NODES]
